# Initial kernel scaffold; baseline (speedup 1.0000x reference)
import jax
import jax.numpy as jnp
from jax.experimental import pallas as pl


def kernel(x, labels, labels2, y, ln_scale, ln_bias, W1, b1, W2, b2):
    raise NotImplementedError("write your pallas kernel here")


# speedup vs baseline: 5.5596x; 5.5596x over previous
import jax
import jax.numpy as jnp
from jax import lax
from jax.experimental import pallas as pl
from jax.experimental.pallas import tpu as pltpu
from jax.experimental.pallas import tpu_sc as plsc

N = 160000
D = 256
YD = 128
L = 1024

B1 = 1280
NB1 = N // B1
G = B1 // 128

NW = 32
T = 128
NTILES = N // T
BASE_Q, BASE_R = NTILES // NW, NTILES % NW
CW = 16
YROWS = 1152


def _stage1(x_ref, lab2_ref, lns_ref, lnb_ref, w1_ref, b1_ref,
            a_ref, base_ref, cum_ref):
    i = pl.program_id(0)

    @pl.when(i == 0)
    def _init():
        cum_ref[0] = 0

    x = x_ref[...]
    mu = jnp.mean(x, axis=1, keepdims=True)
    var = jnp.mean((x - mu) * (x - mu), axis=1, keepdims=True)
    xn = (x - mu) * lax.rsqrt(var + 1e-5)
    xn = xn * lns_ref[...] + lnb_ref[...]
    g = jax.nn.sigmoid(lax.dot(xn, w1_ref[...],
                               preferred_element_type=jnp.float32)
                       + b1_ref[...])
    a_ref[...] = g * xn

    msk = lab2_ref[0]
    gs = jnp.sum(msk, axis=1, keepdims=True).astype(jnp.float32)
    rio = lax.broadcasted_iota(jnp.int32, (G, G), 0)
    cio = lax.broadcasted_iota(jnp.int32, (G, G), 1)
    tri = (cio < rio).astype(jnp.float32)
    excl = lax.dot(tri, gs, preferred_element_type=jnp.float32)
    cum = cum_ref[0]
    base_ref[...] = (cum + excl.astype(jnp.int32)).reshape(1, G, 1)
    cum_ref[0] = cum + jnp.sum(msk)


def _gather16(v, idx):
    return lax.gather(
        v, idx.reshape(16, 1),
        dimension_numbers=lax.GatherDimensionNumbers(
            offset_dims=(), collapsed_slice_dims=(0,), start_index_map=(0,)),
        slice_sizes=(1,),
        mode=lax.GatherScatterMode.PROMISE_IN_BOUNDS)


def _stage2(zacc_hbm, zcnt_hbm, ones_hbm, a_hbm, lab_hbm, lab2_hbm,
            y_hbm, base_hbm,
            acc1_out, acc2_out, yacc_out, cnt_out,
            a1_t, a2_t, y_t, lab_t, lab2_t, seg_t, seg2_t, yidx_t, bvec,
            ones_t, acc1_sh, acc2_sh, yacc_sh, cnt_sh,
            sem_a, sem_a2, sem_y, sem_l, sem_l2, sem_b):
    cid = lax.axis_index("c")
    sid = lax.axis_index("s")
    wid = cid * 16 + sid

    pltpu.sync_copy(zacc_hbm.at[pl.ds(sid * 128, 128), :],
                    acc1_sh.at[pl.ds(sid * 128, 128), :])
    pltpu.sync_copy(zacc_hbm.at[pl.ds(sid * 128, 128), :],
                    acc2_sh.at[pl.ds(sid * 128, 128), :])
    pltpu.sync_copy(zacc_hbm.at[pl.ds(sid * 72, 72), :],
                    yacc_sh.at[pl.ds(sid * 72, 72), :])
    pltpu.sync_copy(zcnt_hbm.at[pl.ds(sid * 128, 128), :],
                    cnt_sh.at[pl.ds(sid * 128, 128), :])
    pltpu.sync_copy(ones_hbm, ones_t)

    extra = jnp.minimum(wid, BASE_R)
    tile0 = BASE_Q * wid + extra
    nt = BASE_Q + (wid < BASE_R).astype(jnp.int32)

    bidx = tile0
    aligned = pl.multiple_of(bidx - lax.rem(bidx, 8), 8)
    pltpu.async_copy(base_hbm.at[pl.ds(aligned, 16)], bvec, sem_b).wait()
    idxv = jnp.full((16,), bidx - aligned, jnp.int32)
    cum0v = _gather16(bvec[...], idxv)

    plsc.subcore_barrier()

    iota16 = lax.broadcasted_iota(jnp.int32, (16,), 0)

    lane15 = jnp.full((16,), 15, jnp.int32)

    def tile_body(t, cumv):
        row0 = pl.multiple_of((tile0 + t) * T, T)
        cp_a = pltpu.async_copy(a_hbm.at[pl.ds(row0, T), pl.ds(0, 128)],
                                a1_t, sem_a)
        cp_a2 = pltpu.async_copy(a_hbm.at[pl.ds(row0, T), pl.ds(128, 128)],
                                 a2_t, sem_a2)
        cp_l = pltpu.async_copy(lab_hbm.at[pl.ds(row0, T)], lab_t, sem_l)
        cp_l2 = pltpu.async_copy(lab2_hbm.at[pl.ds(row0, T)], lab2_t, sem_l2)
        cp_l.wait()
        cp_l2.wait()
        posv = jnp.zeros((16,), jnp.int32)
        for k in range(T // 16):
            sl = pl.ds(k * 16, 16)
            lv = lab_t[sl]
            l2v = lab2_t[sl]
            seg_t[sl] = lv + l2v * L
            seg2_t[sl] = lv * l2v + (1 - l2v) * L
            inc = l2v
            for s in (1, 2, 4, 8):
                shifted = _gather16(inc, jnp.maximum(iota16 - s, 0))
                valid = jnp.minimum(jnp.maximum(iota16 - (s - 1), 0), 1)
                inc = inc + shifted * valid
            yidx_t[sl] = cumv + posv + inc - l2v
            posv = posv + _gather16(inc, lane15)
        cp_y = pltpu.async_copy(y_hbm.at[yidx_t], y_t, sem_y)
        cp_a.wait()
        pltpu.sync_copy(a1_t, acc1_sh.at[seg_t], add=True)
        cp_a2.wait()
        pltpu.sync_copy(a2_t, acc2_sh.at[seg_t], add=True)
        pltpu.sync_copy(ones_t, cnt_sh.at[seg_t], add=True)
        cp_y.wait()
        pltpu.sync_copy(y_t, yacc_sh.at[seg2_t], add=True)
        return cumv + posv

    lax.fori_loop(0, nt, tile_body, cum0v)

    plsc.subcore_barrier()
    pltpu.sync_copy(acc1_sh.at[pl.ds(sid * 128, 128), :],
                    acc1_out.at[cid, pl.ds(sid * 128, 128), :])
    pltpu.sync_copy(acc2_sh.at[pl.ds(sid * 128, 128), :],
                    acc2_out.at[cid, pl.ds(sid * 128, 128), :])
    pltpu.sync_copy(yacc_sh.at[pl.ds(sid * 72, 72), :],
                    yacc_out.at[cid, pl.ds(sid * 72, 72), :])
    pltpu.sync_copy(cnt_sh.at[pl.ds(sid * 128, 128), :],
                    cnt_out.at[cid, pl.ds(sid * 128, 128), :])


def _stage3(acc1_ref, acc2_ref, yacc_ref, cnt_ref, w2_ref, b2_ref, out_ref):
    acc1 = acc1_ref[:2 * L, :] + acc1_ref[2 * L:, :]
    acc2 = acc2_ref[:2 * L, :] + acc2_ref[2 * L:, :]
    cnt = cnt_ref[:2 * L, :1] + cnt_ref[2 * L:, :1]
    c0 = cnt[:L, :]
    c1 = cnt[L:, :]
    diff1 = acc1[L:, :] / c1 - acc1[:L, :] / c0
    diff2 = acc2[L:, :] / c1 - acc2[:L, :] / c0
    logits = (lax.dot(diff1, w2_ref[:128, :],
                      preferred_element_type=jnp.float32)
              + lax.dot(diff2, w2_ref[128:, :],
                        preferred_element_type=jnp.float32) + b2_ref[...])
    mx = jnp.max(logits, axis=1, keepdims=True)
    e = jnp.exp(logits - mx)
    pred = e / jnp.sum(e, axis=1, keepdims=True)
    lse = jnp.log(jnp.sum(jnp.exp(pred), axis=1, keepdims=True))
    logp = pred - lse
    ymean = (yacc_ref[:L, :] + yacc_ref[YROWS:YROWS + L, :]) / c1
    out_ref[...] = (-jnp.sum(ymean * logp) / L) * jnp.ones((1, 1),
                                                           jnp.float32)


@jax.jit
def kernel(x, labels, labels2, y, ln_scale, ln_bias, W1, b1, W2, b2):
    lab = labels.astype(jnp.int32)
    lab2 = labels2.astype(jnp.int32)

    a, base = pl.pallas_call(
        _stage1,
        grid=(NB1,),
        in_specs=[
            pl.BlockSpec((B1, D), lambda i: (i, 0)),
            pl.BlockSpec((1, G, 128), lambda i: (i, 0, 0)),
            pl.BlockSpec((1, D), lambda i: (0, 0)),
            pl.BlockSpec((1, D), lambda i: (0, 0)),
            pl.BlockSpec((D, 1), lambda i: (0, 0)),
            pl.BlockSpec((1, 1), lambda i: (0, 0)),
        ],
        out_specs=[
            pl.BlockSpec((B1, D), lambda i: (i, 0)),
            pl.BlockSpec((1, G, 1), lambda i: (i, 0, 0)),
        ],
        out_shape=[
            jax.ShapeDtypeStruct((N, D), jnp.float32),
            jax.ShapeDtypeStruct((NB1, G, 1), jnp.int32),
        ],
        scratch_shapes=[pltpu.SMEM((1,), jnp.int32)],
    )(x, lab2.reshape(NB1, G, 128), ln_scale.reshape(1, D),
      ln_bias.reshape(1, D), W1, b1.reshape(1, 1))

    base_pad = jnp.concatenate(
        [base.reshape(NTILES), jnp.zeros(16, jnp.int32)])

    sc = pl.kernel(
        _stage2,
        out_type=[
            jax.ShapeDtypeStruct((2, 2 * L, 128), jnp.float32),
            jax.ShapeDtypeStruct((2, 2 * L, 128), jnp.float32),
            jax.ShapeDtypeStruct((2, YROWS, YD), jnp.float32),
            jax.ShapeDtypeStruct((2, 2 * L, CW), jnp.float32),
        ],
        mesh=plsc.VectorSubcoreMesh(core_axis_name="c",
                                    subcore_axis_name="s"),
        scratch_types=[
            pltpu.VMEM((T, 128), jnp.float32),
            pltpu.VMEM((T, 128), jnp.float32),
            pltpu.VMEM((T, YD), jnp.float32),
            pltpu.VMEM((T,), jnp.int32),
            pltpu.VMEM((T,), jnp.int32),
            pltpu.VMEM((T,), jnp.int32),
            pltpu.VMEM((T,), jnp.int32),
            pltpu.VMEM((T,), jnp.int32),
            pltpu.VMEM((16,), jnp.int32),
            pltpu.VMEM((T, CW), jnp.float32),
            pltpu.VMEM_SHARED((2 * L, 128), jnp.float32),
            pltpu.VMEM_SHARED((2 * L, 128), jnp.float32),
            pltpu.VMEM_SHARED((YROWS, YD), jnp.float32),
            pltpu.VMEM_SHARED((2 * L, CW), jnp.float32),
            pltpu.SemaphoreType.DMA,
            pltpu.SemaphoreType.DMA,
            pltpu.SemaphoreType.DMA,
            pltpu.SemaphoreType.DMA,
            pltpu.SemaphoreType.DMA,
            pltpu.SemaphoreType.DMA,
        ],
    )
    accs1, accs2, yaccs, cnts = sc(jnp.zeros((2 * L, 128), jnp.float32),
                                   jnp.zeros((2 * L, CW), jnp.float32),
                                   jnp.ones((T, CW), jnp.float32),
                                   a, lab, lab2, y, base_pad)

    out = pl.pallas_call(
        _stage3,
        in_specs=[
            pl.BlockSpec((4 * L, 128), lambda: (0, 0)),
            pl.BlockSpec((4 * L, 128), lambda: (0, 0)),
            pl.BlockSpec((2 * YROWS, YD), lambda: (0, 0)),
            pl.BlockSpec((4 * L, CW), lambda: (0, 0)),
            pl.BlockSpec((D, YD), lambda: (0, 0)),
            pl.BlockSpec((1, YD), lambda: (0, 0)),
        ],
        out_specs=pl.BlockSpec((1, 1), lambda: (0, 0)),
        out_shape=jax.ShapeDtypeStruct((1, 1), jnp.float32),
    )(accs1.reshape(4 * L, 128), accs2.reshape(4 * L, 128),
      yaccs.reshape(2 * YROWS, YD), cnts.reshape(4 * L, CW), W2,
      b2.reshape(1, YD))
    return out[0, 0]


# speedup vs baseline: 6.6739x; 1.2004x over previous
import jax
import jax.numpy as jnp
from jax import lax
from jax.experimental import pallas as pl
from jax.experimental.pallas import tpu as pltpu
from jax.experimental.pallas import tpu_sc as plsc

N = 160000
D = 256
YD = 128
L = 1024

K = 2
NC = N // K

B1 = 3200
NB1C = NC // B1
G = B1 // 128

NW = 32
T = 128
NTC = NC // T
BASE_Q, BASE_R = NTC // NW, NTC % NW
CW = 16
YROWS = 1152


def _stage1(x_ref, lab2_ref, lns_ref, lnb_ref, w1_ref, b1_ref, cumin_ref,
            a_ref, base_ref, cumout_ref, cum_ref):
    i = pl.program_id(0)

    @pl.when(i == 0)
    def _init():
        cum_ref[0] = cumin_ref[0, 0]

    x = x_ref[...]
    mu = jnp.mean(x, axis=1, keepdims=True)
    var = jnp.mean((x - mu) * (x - mu), axis=1, keepdims=True)
    xn = (x - mu) * lax.rsqrt(var + 1e-5)
    xn = xn * lns_ref[...] + lnb_ref[...]
    g = jax.nn.sigmoid(lax.dot(xn, w1_ref[...],
                               preferred_element_type=jnp.float32)
                       + b1_ref[...])
    a_ref[...] = g * xn

    msk = lab2_ref[0]
    gs = jnp.sum(msk, axis=1, keepdims=True).astype(jnp.float32)
    rio = lax.broadcasted_iota(jnp.int32, (G, G), 0)
    cio = lax.broadcasted_iota(jnp.int32, (G, G), 1)
    tri = (cio < rio).astype(jnp.float32)
    excl = lax.dot(tri, gs, preferred_element_type=jnp.float32)
    cum = cum_ref[0]
    base_ref[...] = (cum + excl.astype(jnp.int32)).reshape(1, G, 1)
    cum_ref[0] = cum + jnp.sum(msk)

    @pl.when(i == NB1C - 1)
    def _fin():
        cumout_ref[...] = cum_ref[0] * jnp.ones((1, 1), jnp.int32)


def _gather16(v, idx):
    return lax.gather(
        v, idx.reshape(16, 1),
        dimension_numbers=lax.GatherDimensionNumbers(
            offset_dims=(), collapsed_slice_dims=(0,), start_index_map=(0,)),
        slice_sizes=(1,),
        mode=lax.GatherScatterMode.PROMISE_IN_BOUNDS)


def _make_stage2(roff):
    def _stage2(zacc_hbm, zcnt_hbm, ones_hbm, a_hbm, lab_hbm, lab2_hbm,
                y_hbm, base_hbm,
                acc1_out, acc2_out, yacc_out, cnt_out,
                a1_t, a2_t, y_t, lab_t, lab2_t, seg_t, seg2_t, yidx_t, bvec,
                ones_t, acc1_sh, acc2_sh, yacc_sh, cnt_sh,
                sem_a, sem_a2, sem_y, sem_l, sem_l2, sem_b):
        cid = lax.axis_index("c")
        sid = lax.axis_index("s")
        wid = cid * 16 + sid

        pltpu.sync_copy(zacc_hbm.at[pl.ds(sid * 128, 128), :],
                        acc1_sh.at[pl.ds(sid * 128, 128), :])
        pltpu.sync_copy(zacc_hbm.at[pl.ds(sid * 128, 128), :],
                        acc2_sh.at[pl.ds(sid * 128, 128), :])
        pltpu.sync_copy(zacc_hbm.at[pl.ds(sid * 72, 72), :],
                        yacc_sh.at[pl.ds(sid * 72, 72), :])
        pltpu.sync_copy(zcnt_hbm.at[pl.ds(sid * 128, 128), :],
                        cnt_sh.at[pl.ds(sid * 128, 128), :])
        pltpu.sync_copy(ones_hbm, ones_t)

        extra = jnp.minimum(wid, BASE_R)
        tile0 = BASE_Q * wid + extra
        nt = BASE_Q + (wid < BASE_R).astype(jnp.int32)

        bidx = tile0
        aligned = pl.multiple_of(bidx - lax.rem(bidx, 8), 8)
        pltpu.async_copy(base_hbm.at[pl.ds(aligned, 16)], bvec, sem_b).wait()
        idxv = jnp.full((16,), bidx - aligned, jnp.int32)
        cum0v = _gather16(bvec[...], idxv)

        plsc.subcore_barrier()

        iota16 = lax.broadcasted_iota(jnp.int32, (16,), 0)
        lane15 = jnp.full((16,), 15, jnp.int32)

        def tile_body(t, cumv):
            row0 = pl.multiple_of((tile0 + t) * T, T)
            cp_a = pltpu.async_copy(a_hbm.at[pl.ds(row0, T), pl.ds(0, 128)],
                                    a1_t, sem_a)
            cp_a2 = pltpu.async_copy(a_hbm.at[pl.ds(row0, T),
                                              pl.ds(128, 128)],
                                     a2_t, sem_a2)
            grow0 = pl.multiple_of(row0 + roff, T)
            cp_l = pltpu.async_copy(lab_hbm.at[pl.ds(grow0, T)], lab_t,
                                    sem_l)
            cp_l2 = pltpu.async_copy(lab2_hbm.at[pl.ds(grow0, T)], lab2_t,
                                     sem_l2)
            cp_l.wait()
            cp_l2.wait()
            posv = jnp.zeros((16,), jnp.int32)
            for k in range(T // 16):
                sl = pl.ds(k * 16, 16)
                lv = lab_t[sl]
                l2v = lab2_t[sl]
                seg_t[sl] = lv + l2v * L
                seg2_t[sl] = lv * l2v + (1 - l2v) * L
                inc = l2v
                for s in (1, 2, 4, 8):
                    shifted = _gather16(inc, jnp.maximum(iota16 - s, 0))
                    valid = jnp.minimum(jnp.maximum(iota16 - (s - 1), 0), 1)
                    inc = inc + shifted * valid
                yidx_t[sl] = cumv + posv + inc - l2v
                posv = posv + _gather16(inc, lane15)
            cp_y = pltpu.async_copy(y_hbm.at[yidx_t], y_t, sem_y)
            cp_a.wait()
            pltpu.sync_copy(a1_t, acc1_sh.at[seg_t], add=True)
            cp_a2.wait()
            pltpu.sync_copy(a2_t, acc2_sh.at[seg_t], add=True)
            pltpu.sync_copy(ones_t, cnt_sh.at[seg_t], add=True)
            cp_y.wait()
            pltpu.sync_copy(y_t, yacc_sh.at[seg2_t], add=True)
            return cumv + posv

        lax.fori_loop(0, nt, tile_body, cum0v)

        plsc.subcore_barrier()
        pltpu.sync_copy(acc1_sh.at[pl.ds(sid * 128, 128), :],
                        acc1_out.at[cid, pl.ds(sid * 128, 128), :])
        pltpu.sync_copy(acc2_sh.at[pl.ds(sid * 128, 128), :],
                        acc2_out.at[cid, pl.ds(sid * 128, 128), :])
        pltpu.sync_copy(yacc_sh.at[pl.ds(sid * 72, 72), :],
                        yacc_out.at[cid, pl.ds(sid * 72, 72), :])
        pltpu.sync_copy(cnt_sh.at[pl.ds(sid * 128, 128), :],
                        cnt_out.at[cid, pl.ds(sid * 128, 128), :])

    return _stage2


def _stage3(acc1a_ref, acc1b_ref, acc2a_ref, acc2b_ref, yacca_ref, yaccb_ref,
            cnta_ref, cntb_ref, w2_ref, b2_ref, out_ref):
    acc1 = (acc1a_ref[:2 * L, :] + acc1a_ref[2 * L:, :]
            + acc1b_ref[:2 * L, :] + acc1b_ref[2 * L:, :])
    acc2 = (acc2a_ref[:2 * L, :] + acc2a_ref[2 * L:, :]
            + acc2b_ref[:2 * L, :] + acc2b_ref[2 * L:, :])
    cnt = (cnta_ref[:2 * L, :1] + cnta_ref[2 * L:, :1]
           + cntb_ref[:2 * L, :1] + cntb_ref[2 * L:, :1])
    c0 = cnt[:L, :]
    c1 = cnt[L:, :]
    diff1 = acc1[L:, :] / c1 - acc1[:L, :] / c0
    diff2 = acc2[L:, :] / c1 - acc2[:L, :] / c0
    logits = (lax.dot(diff1, w2_ref[:128, :],
                      preferred_element_type=jnp.float32)
              + lax.dot(diff2, w2_ref[128:, :],
                        preferred_element_type=jnp.float32) + b2_ref[...])
    mx = jnp.max(logits, axis=1, keepdims=True)
    e = jnp.exp(logits - mx)
    pred = e / jnp.sum(e, axis=1, keepdims=True)
    lse = jnp.log(jnp.sum(jnp.exp(pred), axis=1, keepdims=True))
    logp = pred - lse
    ymean = (yacca_ref[:L, :] + yacca_ref[YROWS:YROWS + L, :]
             + yaccb_ref[:L, :] + yaccb_ref[YROWS:YROWS + L, :]) / c1
    out_ref[...] = (-jnp.sum(ymean * logp) / L) * jnp.ones((1, 1),
                                                           jnp.float32)


@jax.jit
def kernel(x, labels, labels2, y, ln_scale, ln_bias, W1, b1, W2, b2):
    lab = labels.astype(jnp.int32)
    lab2 = labels2.astype(jnp.int32)
    lab2_3d = lab2.reshape(K * NB1C, G, 128)

    zacc = jnp.zeros((2 * L, 128), jnp.float32)
    zcnt = jnp.zeros((2 * L, CW), jnp.float32)
    onesr = jnp.ones((T, CW), jnp.float32)

    def stage1(c, cum_in):
        return pl.pallas_call(
            _stage1,
            grid=(NB1C,),
            in_specs=[
                pl.BlockSpec((B1, D), lambda i, c=c: (i + c * NB1C, 0)),
                pl.BlockSpec((1, G, 128),
                             lambda i, c=c: (i + c * NB1C, 0, 0)),
                pl.BlockSpec((1, D), lambda i: (0, 0)),
                pl.BlockSpec((1, D), lambda i: (0, 0)),
                pl.BlockSpec((D, 1), lambda i: (0, 0)),
                pl.BlockSpec((1, 1), lambda i: (0, 0)),
                pl.BlockSpec((1, 1), lambda i: (0, 0)),
            ],
            out_specs=[
                pl.BlockSpec((B1, D), lambda i: (i, 0)),
                pl.BlockSpec((1, G, 1), lambda i: (i, 0, 0)),
                pl.BlockSpec((1, 1), lambda i: (0, 0)),
            ],
            out_shape=[
                jax.ShapeDtypeStruct((NC, D), jnp.float32),
                jax.ShapeDtypeStruct((NB1C, G, 1), jnp.int32),
                jax.ShapeDtypeStruct((1, 1), jnp.int32),
            ],
            scratch_shapes=[pltpu.SMEM((1,), jnp.int32)],
        )(x, lab2_3d, ln_scale.reshape(1, D), ln_bias.reshape(1, D), W1,
          b1.reshape(1, 1), cum_in)

    def stage2(c, a_c, base_c):
        base_pad = jnp.concatenate(
            [base_c.reshape(NTC), jnp.zeros(16, jnp.int32)])
        f = pl.kernel(
            _make_stage2(c * NC),
            out_type=[
                jax.ShapeDtypeStruct((2, 2 * L, 128), jnp.float32),
                jax.ShapeDtypeStruct((2, 2 * L, 128), jnp.float32),
                jax.ShapeDtypeStruct((2, YROWS, YD), jnp.float32),
                jax.ShapeDtypeStruct((2, 2 * L, CW), jnp.float32),
            ],
            mesh=plsc.VectorSubcoreMesh(core_axis_name="c",
                                        subcore_axis_name="s"),
            scratch_types=[
                pltpu.VMEM((T, 128), jnp.float32),
                pltpu.VMEM((T, 128), jnp.float32),
                pltpu.VMEM((T, YD), jnp.float32),
                pltpu.VMEM((T,), jnp.int32),
                pltpu.VMEM((T,), jnp.int32),
                pltpu.VMEM((T,), jnp.int32),
                pltpu.VMEM((T,), jnp.int32),
                pltpu.VMEM((T,), jnp.int32),
                pltpu.VMEM((16,), jnp.int32),
                pltpu.VMEM((T, CW), jnp.float32),
                pltpu.VMEM_SHARED((2 * L, 128), jnp.float32),
                pltpu.VMEM_SHARED((2 * L, 128), jnp.float32),
                pltpu.VMEM_SHARED((YROWS, YD), jnp.float32),
                pltpu.VMEM_SHARED((2 * L, CW), jnp.float32),
                pltpu.SemaphoreType.DMA,
                pltpu.SemaphoreType.DMA,
                pltpu.SemaphoreType.DMA,
                pltpu.SemaphoreType.DMA,
                pltpu.SemaphoreType.DMA,
                pltpu.SemaphoreType.DMA,
            ],
        )
        return f(zacc, zcnt, onesr, a_c, lab, lab2, y, base_pad)

    a0, base0, cum1 = stage1(0, jnp.zeros((1, 1), jnp.int32))
    a1, base1, _ = stage1(1, cum1)
    acc1a, acc2a, yacca, cnta = stage2(0, a0, base0)
    acc1b, acc2b, yaccb, cntb = stage2(1, a1, base1)

    out = pl.pallas_call(
        _stage3,
        in_specs=[
            pl.BlockSpec((4 * L, 128), lambda: (0, 0)),
            pl.BlockSpec((4 * L, 128), lambda: (0, 0)),
            pl.BlockSpec((4 * L, 128), lambda: (0, 0)),
            pl.BlockSpec((4 * L, 128), lambda: (0, 0)),
            pl.BlockSpec((2 * YROWS, YD), lambda: (0, 0)),
            pl.BlockSpec((2 * YROWS, YD), lambda: (0, 0)),
            pl.BlockSpec((4 * L, CW), lambda: (0, 0)),
            pl.BlockSpec((4 * L, CW), lambda: (0, 0)),
            pl.BlockSpec((D, YD), lambda: (0, 0)),
            pl.BlockSpec((1, YD), lambda: (0, 0)),
        ],
        out_specs=pl.BlockSpec((1, 1), lambda: (0, 0)),
        out_shape=jax.ShapeDtypeStruct((1, 1), jnp.float32),
    )(acc1a.reshape(4 * L, 128), acc1b.reshape(4 * L, 128),
      acc2a.reshape(4 * L, 128), acc2b.reshape(4 * L, 128),
      yacca.reshape(2 * YROWS, YD), yaccb.reshape(2 * YROWS, YD),
      cnta.reshape(4 * L, CW), cntb.reshape(4 * L, CW), W2,
      b2.reshape(1, YD))
    return out[0, 0]


# speedup vs baseline: 7.0112x; 1.0505x over previous
import jax
import jax.numpy as jnp
from jax import lax
from jax.experimental import pallas as pl
from jax.experimental.pallas import tpu as pltpu
from jax.experimental.pallas import tpu_sc as plsc

N = 160000
D = 256
YD = 128
L = 1024

K = 2
NC = N // K

B1 = 3200
NB1C = NC // B1
G = B1 // 128

NW = 32
T = 128
NTC = NC // T
BASE_Q, BASE_R = NTC // NW, NTC % NW
CW = 16
YROWS = 1152


def _stage1(x_ref, lab2_ref, lns_ref, lnb_ref, w1_ref, b1_ref, cumin_ref,
            a_ref, base_ref, cumout_ref, cum_ref):
    i = pl.program_id(0)

    @pl.when(i == 0)
    def _init():
        cum_ref[0] = cumin_ref[0, 0]

    x = x_ref[...]
    mu = jnp.mean(x, axis=1, keepdims=True)
    var = jnp.mean((x - mu) * (x - mu), axis=1, keepdims=True)
    xn = (x - mu) * lax.rsqrt(var + 1e-5)
    xn = xn * lns_ref[...] + lnb_ref[...]
    g = jax.nn.sigmoid(lax.dot(xn, w1_ref[...],
                               preferred_element_type=jnp.float32)
                       + b1_ref[...])
    a_ref[...] = g * xn

    msk = lab2_ref[0]
    gs = jnp.sum(msk, axis=1, keepdims=True).astype(jnp.float32)
    rio = lax.broadcasted_iota(jnp.int32, (G, G), 0)
    cio = lax.broadcasted_iota(jnp.int32, (G, G), 1)
    tri = (cio < rio).astype(jnp.float32)
    excl = lax.dot(tri, gs, preferred_element_type=jnp.float32)
    cum = cum_ref[0]
    base_ref[...] = (cum + excl.astype(jnp.int32)).reshape(1, G, 1)
    cum_ref[0] = cum + jnp.sum(msk)

    @pl.when(i == NB1C - 1)
    def _fin():
        cumout_ref[...] = cum_ref[0] * jnp.ones((1, 1), jnp.int32)


def _gather16(v, idx):
    return lax.gather(
        v, idx.reshape(16, 1),
        dimension_numbers=lax.GatherDimensionNumbers(
            offset_dims=(), collapsed_slice_dims=(0,), start_index_map=(0,)),
        slice_sizes=(1,),
        mode=lax.GatherScatterMode.PROMISE_IN_BOUNDS)


def _make_stage2(roff):
    def _stage2(zacc_hbm, zcnt_hbm, ones_hbm, a_hbm, lab_hbm, lab2_hbm,
                y_hbm, base_hbm,
                acc1_out, acc2_out, yacc_out, cnt_out,
                a1_t, a2_t, y_t, lab_t, lab2_t, seg_t, seg2_t, yidx_t, bvec,
                ones_t, acc1_sh, acc2_sh, yacc_sh, cnt_sh,
                sem_a, sem_a2, sem_y, sem_l, sem_l2, sem_b):
        cid = lax.axis_index("c")
        sid = lax.axis_index("s")
        wid = cid * 16 + sid

        pltpu.sync_copy(zacc_hbm.at[pl.ds(sid * 128, 128), :],
                        acc1_sh.at[pl.ds(sid * 128, 128), :])
        pltpu.sync_copy(zacc_hbm.at[pl.ds(sid * 128, 128), :],
                        acc2_sh.at[pl.ds(sid * 128, 128), :])
        pltpu.sync_copy(zacc_hbm.at[pl.ds(sid * 72, 72), :],
                        yacc_sh.at[pl.ds(sid * 72, 72), :])
        pltpu.sync_copy(zcnt_hbm.at[pl.ds(sid * 128, 128), :],
                        cnt_sh.at[pl.ds(sid * 128, 128), :])
        pltpu.sync_copy(ones_hbm, ones_t)

        extra = jnp.minimum(wid, BASE_R)
        tile0 = BASE_Q * wid + extra
        nt = BASE_Q + (wid < BASE_R).astype(jnp.int32)

        bidx = tile0
        aligned = pl.multiple_of(bidx - lax.rem(bidx, 8), 8)
        pltpu.async_copy(base_hbm.at[pl.ds(aligned, 16)], bvec, sem_b).wait()
        idxv = jnp.full((16,), bidx - aligned, jnp.int32)
        cum0v = _gather16(bvec[...], idxv)

        plsc.subcore_barrier()

        iota16 = lax.broadcasted_iota(jnp.int32, (16,), 0)
        lane15 = jnp.full((16,), 15, jnp.int32)

        def tile_body(t, cumv):
            row0 = pl.multiple_of((tile0 + t) * T, T)
            cp_a = pltpu.async_copy(a_hbm.at[pl.ds(row0, T), pl.ds(0, 128)],
                                    a1_t, sem_a)
            cp_a2 = pltpu.async_copy(a_hbm.at[pl.ds(row0, T),
                                              pl.ds(128, 128)],
                                     a2_t, sem_a2)
            grow0 = pl.multiple_of(row0 + roff, T)
            cp_l = pltpu.async_copy(lab_hbm.at[pl.ds(grow0, T)], lab_t,
                                    sem_l)
            cp_l2 = pltpu.async_copy(lab2_hbm.at[pl.ds(grow0, T)], lab2_t,
                                     sem_l2)
            cp_l.wait()
            cp_l2.wait()
            posv = jnp.zeros((16,), jnp.int32)
            for k in range(T // 16):
                sl = pl.ds(k * 16, 16)
                lv = lab_t[sl]
                l2v = lab2_t[sl]
                seg_t[sl] = lv + l2v * L
                seg2_t[sl] = lv * l2v + (1 - l2v) * L
                inc = l2v
                for s in (1, 2, 4, 8):
                    shifted = _gather16(inc, jnp.maximum(iota16 - s, 0))
                    valid = jnp.minimum(jnp.maximum(iota16 - (s - 1), 0), 1)
                    inc = inc + shifted * valid
                rnk = cumv + posv + inc - l2v
                yidx_t[sl] = rnk * l2v + (1 - l2v) * N
                posv = posv + _gather16(inc, lane15)
            cp_y = pltpu.async_copy(
                y_hbm.at[plsc.Indices(yidx_t, ignored_value=N)], y_t, sem_y)
            cp_a.wait()
            pltpu.sync_copy(a1_t, acc1_sh.at[seg_t], add=True)
            cp_a2.wait()
            pltpu.sync_copy(a2_t, acc2_sh.at[seg_t], add=True)
            pltpu.sync_copy(ones_t, cnt_sh.at[seg_t], add=True)
            cp_y.wait()
            pltpu.sync_copy(
                y_t, yacc_sh.at[plsc.Indices(seg2_t, ignored_value=L)],
                add=True)
            return cumv + posv

        lax.fori_loop(0, nt, tile_body, cum0v)

        plsc.subcore_barrier()
        pltpu.sync_copy(acc1_sh.at[pl.ds(sid * 128, 128), :],
                        acc1_out.at[cid, pl.ds(sid * 128, 128), :])
        pltpu.sync_copy(acc2_sh.at[pl.ds(sid * 128, 128), :],
                        acc2_out.at[cid, pl.ds(sid * 128, 128), :])
        pltpu.sync_copy(yacc_sh.at[pl.ds(sid * 72, 72), :],
                        yacc_out.at[cid, pl.ds(sid * 72, 72), :])
        pltpu.sync_copy(cnt_sh.at[pl.ds(sid * 128, 128), :],
                        cnt_out.at[cid, pl.ds(sid * 128, 128), :])

    return _stage2


def _stage3(acc1a_ref, acc1b_ref, acc2a_ref, acc2b_ref, yacca_ref, yaccb_ref,
            cnta_ref, cntb_ref, w2_ref, b2_ref, out_ref):
    acc1 = (acc1a_ref[:2 * L, :] + acc1a_ref[2 * L:, :]
            + acc1b_ref[:2 * L, :] + acc1b_ref[2 * L:, :])
    acc2 = (acc2a_ref[:2 * L, :] + acc2a_ref[2 * L:, :]
            + acc2b_ref[:2 * L, :] + acc2b_ref[2 * L:, :])
    cnt = (cnta_ref[:2 * L, :1] + cnta_ref[2 * L:, :1]
           + cntb_ref[:2 * L, :1] + cntb_ref[2 * L:, :1])
    c0 = cnt[:L, :]
    c1 = cnt[L:, :]
    diff1 = acc1[L:, :] / c1 - acc1[:L, :] / c0
    diff2 = acc2[L:, :] / c1 - acc2[:L, :] / c0
    logits = (lax.dot(diff1, w2_ref[:128, :],
                      preferred_element_type=jnp.float32)
              + lax.dot(diff2, w2_ref[128:, :],
                        preferred_element_type=jnp.float32) + b2_ref[...])
    mx = jnp.max(logits, axis=1, keepdims=True)
    e = jnp.exp(logits - mx)
    pred = e / jnp.sum(e, axis=1, keepdims=True)
    lse = jnp.log(jnp.sum(jnp.exp(pred), axis=1, keepdims=True))
    logp = pred - lse
    ymean = (yacca_ref[:L, :] + yacca_ref[YROWS:YROWS + L, :]
             + yaccb_ref[:L, :] + yaccb_ref[YROWS:YROWS + L, :]) / c1
    out_ref[...] = (-jnp.sum(ymean * logp) / L) * jnp.ones((1, 1),
                                                           jnp.float32)


@jax.jit
def kernel(x, labels, labels2, y, ln_scale, ln_bias, W1, b1, W2, b2):
    lab = labels.astype(jnp.int32)
    lab2 = labels2.astype(jnp.int32)
    lab2_3d = lab2.reshape(K * NB1C, G, 128)

    zacc = jnp.zeros((2 * L, 128), jnp.float32)
    zcnt = jnp.zeros((2 * L, CW), jnp.float32)
    onesr = jnp.ones((T, CW), jnp.float32)

    def stage1(c, cum_in):
        return pl.pallas_call(
            _stage1,
            grid=(NB1C,),
            in_specs=[
                pl.BlockSpec((B1, D), lambda i, c=c: (i + c * NB1C, 0)),
                pl.BlockSpec((1, G, 128),
                             lambda i, c=c: (i + c * NB1C, 0, 0)),
                pl.BlockSpec((1, D), lambda i: (0, 0)),
                pl.BlockSpec((1, D), lambda i: (0, 0)),
                pl.BlockSpec((D, 1), lambda i: (0, 0)),
                pl.BlockSpec((1, 1), lambda i: (0, 0)),
                pl.BlockSpec((1, 1), lambda i: (0, 0)),
            ],
            out_specs=[
                pl.BlockSpec((B1, D), lambda i: (i, 0)),
                pl.BlockSpec((1, G, 1), lambda i: (i, 0, 0)),
                pl.BlockSpec((1, 1), lambda i: (0, 0)),
            ],
            out_shape=[
                jax.ShapeDtypeStruct((NC, D), jnp.float32),
                jax.ShapeDtypeStruct((NB1C, G, 1), jnp.int32),
                jax.ShapeDtypeStruct((1, 1), jnp.int32),
            ],
            scratch_shapes=[pltpu.SMEM((1,), jnp.int32)],
        )(x, lab2_3d, ln_scale.reshape(1, D), ln_bias.reshape(1, D), W1,
          b1.reshape(1, 1), cum_in)

    def stage2(c, a_c, base_c):
        base_pad = jnp.concatenate(
            [base_c.reshape(NTC), jnp.zeros(16, jnp.int32)])
        f = pl.kernel(
            _make_stage2(c * NC),
            out_type=[
                jax.ShapeDtypeStruct((2, 2 * L, 128), jnp.float32),
                jax.ShapeDtypeStruct((2, 2 * L, 128), jnp.float32),
                jax.ShapeDtypeStruct((2, YROWS, YD), jnp.float32),
                jax.ShapeDtypeStruct((2, 2 * L, CW), jnp.float32),
            ],
            mesh=plsc.VectorSubcoreMesh(core_axis_name="c",
                                        subcore_axis_name="s"),
            scratch_types=[
                pltpu.VMEM((T, 128), jnp.float32),
                pltpu.VMEM((T, 128), jnp.float32),
                pltpu.VMEM((T, YD), jnp.float32),
                pltpu.VMEM((T,), jnp.int32),
                pltpu.VMEM((T,), jnp.int32),
                pltpu.VMEM((T,), jnp.int32),
                pltpu.VMEM((T,), jnp.int32),
                pltpu.VMEM((T,), jnp.int32),
                pltpu.VMEM((16,), jnp.int32),
                pltpu.VMEM((T, CW), jnp.float32),
                pltpu.VMEM_SHARED((2 * L, 128), jnp.float32),
                pltpu.VMEM_SHARED((2 * L, 128), jnp.float32),
                pltpu.VMEM_SHARED((YROWS, YD), jnp.float32),
                pltpu.VMEM_SHARED((2 * L, CW), jnp.float32),
                pltpu.SemaphoreType.DMA,
                pltpu.SemaphoreType.DMA,
                pltpu.SemaphoreType.DMA,
                pltpu.SemaphoreType.DMA,
                pltpu.SemaphoreType.DMA,
                pltpu.SemaphoreType.DMA,
            ],
        )
        return f(zacc, zcnt, onesr, a_c, lab, lab2, y, base_pad)

    a0, base0, cum1 = stage1(0, jnp.zeros((1, 1), jnp.int32))
    a1, base1, _ = stage1(1, cum1)
    acc1a, acc2a, yacca, cnta = stage2(0, a0, base0)
    acc1b, acc2b, yaccb, cntb = stage2(1, a1, base1)

    out = pl.pallas_call(
        _stage3,
        in_specs=[
            pl.BlockSpec((4 * L, 128), lambda: (0, 0)),
            pl.BlockSpec((4 * L, 128), lambda: (0, 0)),
            pl.BlockSpec((4 * L, 128), lambda: (0, 0)),
            pl.BlockSpec((4 * L, 128), lambda: (0, 0)),
            pl.BlockSpec((2 * YROWS, YD), lambda: (0, 0)),
            pl.BlockSpec((2 * YROWS, YD), lambda: (0, 0)),
            pl.BlockSpec((4 * L, CW), lambda: (0, 0)),
            pl.BlockSpec((4 * L, CW), lambda: (0, 0)),
            pl.BlockSpec((D, YD), lambda: (0, 0)),
            pl.BlockSpec((1, YD), lambda: (0, 0)),
        ],
        out_specs=pl.BlockSpec((1, 1), lambda: (0, 0)),
        out_shape=jax.ShapeDtypeStruct((1, 1), jnp.float32),
    )(acc1a.reshape(4 * L, 128), acc1b.reshape(4 * L, 128),
      acc2a.reshape(4 * L, 128), acc2b.reshape(4 * L, 128),
      yacca.reshape(2 * YROWS, YD), yaccb.reshape(2 * YROWS, YD),
      cnta.reshape(4 * L, CW), cntb.reshape(4 * L, CW), W2,
      b2.reshape(1, YD))
    return out[0, 0]


# speedup vs baseline: 7.1516x; 1.0200x over previous
import jax
import jax.numpy as jnp
from jax import lax
from jax.experimental import pallas as pl
from jax.experimental.pallas import tpu as pltpu
from jax.experimental.pallas import tpu_sc as plsc

N = 160000
D = 256
YD = 128
L = 1024

K = 2
NC = N // K

B1 = 3200
NB1C = NC // B1
G = B1 // 128

NW = 32
T = 128
NTC = NC // T
BASE_Q, BASE_R = NTC // NW, NTC % NW
CW = 16
YROWS = 1152


def _stage1(x_ref, lab2_ref, lns_ref, lnb_ref, w1_ref, b1_ref, cumin_ref,
            a_ref, base_ref, cumout_ref, cum_ref):
    i = pl.program_id(0)

    @pl.when(i == 0)
    def _init():
        cum_ref[0] = cumin_ref[0, 0]

    x = x_ref[...]
    mu = jnp.mean(x, axis=1, keepdims=True)
    var = jnp.mean((x - mu) * (x - mu), axis=1, keepdims=True)
    xn = (x - mu) * lax.rsqrt(var + 1e-5)
    xn = xn * lns_ref[...] + lnb_ref[...]
    g = jax.nn.sigmoid(lax.dot(xn, w1_ref[...],
                               preferred_element_type=jnp.float32)
                       + b1_ref[...])
    a_ref[...] = g * xn

    msk = lab2_ref[0]
    gs = jnp.sum(msk, axis=1, keepdims=True).astype(jnp.float32)
    rio = lax.broadcasted_iota(jnp.int32, (G, G), 0)
    cio = lax.broadcasted_iota(jnp.int32, (G, G), 1)
    tri = (cio < rio).astype(jnp.float32)
    excl = lax.dot(tri, gs, preferred_element_type=jnp.float32)
    cum = cum_ref[0]
    base_ref[...] = (cum + excl.astype(jnp.int32)).reshape(1, G, 1)
    cum_ref[0] = cum + jnp.sum(msk)

    @pl.when(i == NB1C - 1)
    def _fin():
        cumout_ref[...] = cum_ref[0] * jnp.ones((1, 1), jnp.int32)


def _gather16(v, idx):
    return lax.gather(
        v, idx.reshape(16, 1),
        dimension_numbers=lax.GatherDimensionNumbers(
            offset_dims=(), collapsed_slice_dims=(0,), start_index_map=(0,)),
        slice_sizes=(1,),
        mode=lax.GatherScatterMode.PROMISE_IN_BOUNDS)


def _make_stage2(roff):
    def _stage2(zacc_hbm, zcnt_hbm, ones_hbm, a_hbm, lab_hbm, lab2_hbm,
                y_hbm, base_hbm,
                acc1_out, acc2_out, yacc_out, cnt_out,
                a1_t, a2_t, y_t, lab_t, lab2_t, seg_t, seg2_t, yidx_t, bvec,
                ones_t, acc1_sh, acc2_sh, yacc_sh, cnt_sh,
                sem_a, sem_a2, sem_y, sem_l, sem_l2, sem_b,
                sem_s1, sem_s2, sem_s3, sem_s4):
        cid = lax.axis_index("c")
        sid = lax.axis_index("s")
        wid = cid * 16 + sid

        pltpu.sync_copy(zacc_hbm.at[pl.ds(sid * 128, 128), :],
                        acc1_sh.at[pl.ds(sid * 128, 128), :])
        pltpu.sync_copy(zacc_hbm.at[pl.ds(sid * 128, 128), :],
                        acc2_sh.at[pl.ds(sid * 128, 128), :])
        pltpu.sync_copy(zacc_hbm.at[pl.ds(sid * 72, 72), :],
                        yacc_sh.at[pl.ds(sid * 72, 72), :])
        pltpu.sync_copy(zcnt_hbm.at[pl.ds(sid * 128, 128), :],
                        cnt_sh.at[pl.ds(sid * 128, 128), :])
        pltpu.sync_copy(ones_hbm, ones_t)

        extra = jnp.minimum(wid, BASE_R)
        tile0 = BASE_Q * wid + extra
        nt = BASE_Q + (wid < BASE_R).astype(jnp.int32)

        bidx = tile0
        aligned = pl.multiple_of(bidx - lax.rem(bidx, 8), 8)
        pltpu.async_copy(base_hbm.at[pl.ds(aligned, 16)], bvec, sem_b).wait()
        idxv = jnp.full((16,), bidx - aligned, jnp.int32)
        cum0v = _gather16(bvec[...], idxv)

        plsc.subcore_barrier()

        iota16 = lax.broadcasted_iota(jnp.int32, (16,), 0)
        lane15 = jnp.full((16,), 15, jnp.int32)

        def tile_body(t, cumv):
            row0 = pl.multiple_of((tile0 + t) * T, T)
            cp_a = pltpu.async_copy(a_hbm.at[pl.ds(row0, T), pl.ds(0, 128)],
                                    a1_t, sem_a)
            cp_a2 = pltpu.async_copy(a_hbm.at[pl.ds(row0, T),
                                              pl.ds(128, 128)],
                                     a2_t, sem_a2)
            grow0 = pl.multiple_of(row0 + roff, T)
            cp_l = pltpu.async_copy(lab_hbm.at[pl.ds(grow0, T)], lab_t,
                                    sem_l)
            cp_l2 = pltpu.async_copy(lab2_hbm.at[pl.ds(grow0, T)], lab2_t,
                                     sem_l2)
            cp_l.wait()
            cp_l2.wait()
            posv = jnp.zeros((16,), jnp.int32)
            for k in range(T // 16):
                sl = pl.ds(k * 16, 16)
                lv = lab_t[sl]
                l2v = lab2_t[sl]
                seg_t[sl] = lv + l2v * L
                seg2_t[sl] = lv * l2v + (1 - l2v) * L
                inc = l2v
                for s in (1, 2, 4, 8):
                    shifted = _gather16(inc, jnp.maximum(iota16 - s, 0))
                    valid = jnp.minimum(jnp.maximum(iota16 - (s - 1), 0), 1)
                    inc = inc + shifted * valid
                rnk = cumv + posv + inc - l2v
                yidx_t[sl] = rnk * l2v + (1 - l2v) * N
                posv = posv + _gather16(inc, lane15)
            cp_y = pltpu.async_copy(
                y_hbm.at[plsc.Indices(yidx_t, ignored_value=N)], y_t, sem_y)
            cp_a.wait()
            s1 = pltpu.async_copy(a1_t, acc1_sh.at[seg_t], sem_s1,
                                  add=True)
            cp_a2.wait()
            s2 = pltpu.async_copy(a2_t, acc2_sh.at[seg_t], sem_s2,
                                  add=True)
            s3 = pltpu.async_copy(ones_t, cnt_sh.at[seg_t], sem_s3,
                                  add=True)
            cp_y.wait()
            s4 = pltpu.async_copy(
                y_t, yacc_sh.at[plsc.Indices(seg2_t, ignored_value=L)],
                sem_s4, add=True)
            s1.wait()
            s2.wait()
            s3.wait()
            s4.wait()
            return cumv + posv

        lax.fori_loop(0, nt, tile_body, cum0v)

        plsc.subcore_barrier()
        pltpu.sync_copy(acc1_sh.at[pl.ds(sid * 128, 128), :],
                        acc1_out.at[cid, pl.ds(sid * 128, 128), :])
        pltpu.sync_copy(acc2_sh.at[pl.ds(sid * 128, 128), :],
                        acc2_out.at[cid, pl.ds(sid * 128, 128), :])
        pltpu.sync_copy(yacc_sh.at[pl.ds(sid * 72, 72), :],
                        yacc_out.at[cid, pl.ds(sid * 72, 72), :])
        pltpu.sync_copy(cnt_sh.at[pl.ds(sid * 128, 128), :],
                        cnt_out.at[cid, pl.ds(sid * 128, 128), :])

    return _stage2


def _stage3(acc1a_ref, acc1b_ref, acc2a_ref, acc2b_ref, yacca_ref, yaccb_ref,
            cnta_ref, cntb_ref, w2_ref, b2_ref, out_ref):
    acc1 = (acc1a_ref[:2 * L, :] + acc1a_ref[2 * L:, :]
            + acc1b_ref[:2 * L, :] + acc1b_ref[2 * L:, :])
    acc2 = (acc2a_ref[:2 * L, :] + acc2a_ref[2 * L:, :]
            + acc2b_ref[:2 * L, :] + acc2b_ref[2 * L:, :])
    cnt = (cnta_ref[:2 * L, :1] + cnta_ref[2 * L:, :1]
           + cntb_ref[:2 * L, :1] + cntb_ref[2 * L:, :1])
    c0 = cnt[:L, :]
    c1 = cnt[L:, :]
    diff1 = acc1[L:, :] / c1 - acc1[:L, :] / c0
    diff2 = acc2[L:, :] / c1 - acc2[:L, :] / c0
    logits = (lax.dot(diff1, w2_ref[:128, :],
                      preferred_element_type=jnp.float32)
              + lax.dot(diff2, w2_ref[128:, :],
                        preferred_element_type=jnp.float32) + b2_ref[...])
    mx = jnp.max(logits, axis=1, keepdims=True)
    e = jnp.exp(logits - mx)
    pred = e / jnp.sum(e, axis=1, keepdims=True)
    lse = jnp.log(jnp.sum(jnp.exp(pred), axis=1, keepdims=True))
    logp = pred - lse
    ymean = (yacca_ref[:L, :] + yacca_ref[YROWS:YROWS + L, :]
             + yaccb_ref[:L, :] + yaccb_ref[YROWS:YROWS + L, :]) / c1
    out_ref[...] = (-jnp.sum(ymean * logp) / L) * jnp.ones((1, 1),
                                                           jnp.float32)


@jax.jit
def kernel(x, labels, labels2, y, ln_scale, ln_bias, W1, b1, W2, b2):
    lab = labels.astype(jnp.int32)
    lab2 = labels2.astype(jnp.int32)
    lab2_3d = lab2.reshape(K * NB1C, G, 128)

    zacc = jnp.zeros((2 * L, 128), jnp.float32)
    zcnt = jnp.zeros((2 * L, CW), jnp.float32)
    onesr = jnp.ones((T, CW), jnp.float32)

    def stage1(c, cum_in):
        return pl.pallas_call(
            _stage1,
            grid=(NB1C,),
            in_specs=[
                pl.BlockSpec((B1, D), lambda i, c=c: (i + c * NB1C, 0)),
                pl.BlockSpec((1, G, 128),
                             lambda i, c=c: (i + c * NB1C, 0, 0)),
                pl.BlockSpec((1, D), lambda i: (0, 0)),
                pl.BlockSpec((1, D), lambda i: (0, 0)),
                pl.BlockSpec((D, 1), lambda i: (0, 0)),
                pl.BlockSpec((1, 1), lambda i: (0, 0)),
                pl.BlockSpec((1, 1), lambda i: (0, 0)),
            ],
            out_specs=[
                pl.BlockSpec((B1, D), lambda i: (i, 0)),
                pl.BlockSpec((1, G, 1), lambda i: (i, 0, 0)),
                pl.BlockSpec((1, 1), lambda i: (0, 0)),
            ],
            out_shape=[
                jax.ShapeDtypeStruct((NC, D), jnp.float32),
                jax.ShapeDtypeStruct((NB1C, G, 1), jnp.int32),
                jax.ShapeDtypeStruct((1, 1), jnp.int32),
            ],
            scratch_shapes=[pltpu.SMEM((1,), jnp.int32)],
        )(x, lab2_3d, ln_scale.reshape(1, D), ln_bias.reshape(1, D), W1,
          b1.reshape(1, 1), cum_in)

    def stage2(c, a_c, base_c):
        base_pad = jnp.concatenate(
            [base_c.reshape(NTC), jnp.zeros(16, jnp.int32)])
        f = pl.kernel(
            _make_stage2(c * NC),
            out_type=[
                jax.ShapeDtypeStruct((2, 2 * L, 128), jnp.float32),
                jax.ShapeDtypeStruct((2, 2 * L, 128), jnp.float32),
                jax.ShapeDtypeStruct((2, YROWS, YD), jnp.float32),
                jax.ShapeDtypeStruct((2, 2 * L, CW), jnp.float32),
            ],
            mesh=plsc.VectorSubcoreMesh(core_axis_name="c",
                                        subcore_axis_name="s"),
            scratch_types=[
                pltpu.VMEM((T, 128), jnp.float32),
                pltpu.VMEM((T, 128), jnp.float32),
                pltpu.VMEM((T, YD), jnp.float32),
                pltpu.VMEM((T,), jnp.int32),
                pltpu.VMEM((T,), jnp.int32),
                pltpu.VMEM((T,), jnp.int32),
                pltpu.VMEM((T,), jnp.int32),
                pltpu.VMEM((T,), jnp.int32),
                pltpu.VMEM((16,), jnp.int32),
                pltpu.VMEM((T, CW), jnp.float32),
                pltpu.VMEM_SHARED((2 * L, 128), jnp.float32),
                pltpu.VMEM_SHARED((2 * L, 128), jnp.float32),
                pltpu.VMEM_SHARED((YROWS, YD), jnp.float32),
                pltpu.VMEM_SHARED((2 * L, CW), jnp.float32),
                pltpu.SemaphoreType.DMA,
                pltpu.SemaphoreType.DMA,
                pltpu.SemaphoreType.DMA,
                pltpu.SemaphoreType.DMA,
                pltpu.SemaphoreType.DMA,
                pltpu.SemaphoreType.DMA,
                pltpu.SemaphoreType.DMA,
                pltpu.SemaphoreType.DMA,
                pltpu.SemaphoreType.DMA,
                pltpu.SemaphoreType.DMA,
            ],
        )
        return f(zacc, zcnt, onesr, a_c, lab, lab2, y, base_pad)

    a0, base0, cum1 = stage1(0, jnp.zeros((1, 1), jnp.int32))
    a1, base1, _ = stage1(1, cum1)
    acc1a, acc2a, yacca, cnta = stage2(0, a0, base0)
    acc1b, acc2b, yaccb, cntb = stage2(1, a1, base1)

    out = pl.pallas_call(
        _stage3,
        in_specs=[
            pl.BlockSpec((4 * L, 128), lambda: (0, 0)),
            pl.BlockSpec((4 * L, 128), lambda: (0, 0)),
            pl.BlockSpec((4 * L, 128), lambda: (0, 0)),
            pl.BlockSpec((4 * L, 128), lambda: (0, 0)),
            pl.BlockSpec((2 * YROWS, YD), lambda: (0, 0)),
            pl.BlockSpec((2 * YROWS, YD), lambda: (0, 0)),
            pl.BlockSpec((4 * L, CW), lambda: (0, 0)),
            pl.BlockSpec((4 * L, CW), lambda: (0, 0)),
            pl.BlockSpec((D, YD), lambda: (0, 0)),
            pl.BlockSpec((1, YD), lambda: (0, 0)),
        ],
        out_specs=pl.BlockSpec((1, 1), lambda: (0, 0)),
        out_shape=jax.ShapeDtypeStruct((1, 1), jnp.float32),
    )(acc1a.reshape(4 * L, 128), acc1b.reshape(4 * L, 128),
      acc2a.reshape(4 * L, 128), acc2b.reshape(4 * L, 128),
      yacca.reshape(2 * YROWS, YD), yaccb.reshape(2 * YROWS, YD),
      cnta.reshape(4 * L, CW), cntb.reshape(4 * L, CW), W2,
      b2.reshape(1, YD))
    return out[0, 0]
